# Initial kernel scaffold; baseline (speedup 1.0000x reference)
import jax
import jax.numpy as jnp
from jax.experimental import pallas as pl
from jax.experimental.pallas import tpu as pltpu

_ROWS = 512


def _body(x_ref, y_ref, wT_ref, invw_ref, esc_ref, df_ref, erep_ref,
          sseg_ref, erep2_ref, giota_ref, dq_ref, q0_ref, out_ref):
    f32 = jnp.float32
    x = x_ref[...]
    wT = wT_ref[...]
    Xm = jax.lax.dot_general(x, wT, (((1,), (0,)), ((), ())),
                             preferred_element_type=f32)
    ndom = invw_ref.shape[0]
    oh = (y_ref[...] == jax.lax.broadcasted_iota(jnp.int32, (1, ndom), 1)
          ).astype(f32)
    xr = jax.lax.dot_general(Xm, erep_ref[...], (((1,), (0,)), ((), ())),
                             preferred_element_type=f32)
    invw_s = jax.lax.dot_general(oh, invw_ref[...], (((1,), (0,)), ((), ())),
                                 preferred_element_type=f32)
    esc_s = jax.lax.dot_general(oh, esc_ref[...], (((1,), (0,)), ((), ())),
                                preferred_element_type=f32)
    df_s = jax.lax.dot_general(oh, df_ref[...], (((1,), (0,)), ((), ())),
                               preferred_element_type=f32)
    A = jnp.clip(xr * invw_s - esc_s, 0.0, 1.0)
    u = jax.lax.dot_general(df_s * A, sseg_ref[...], (((1,), (0,)), ((), ())),
                            preferred_element_type=f32)
    ur = jax.lax.dot_general(u, erep2_ref[...], (((1,), (0,)), ((), ())),
                             preferred_element_type=f32)
    A2 = jnp.clip(ur - giota_ref[...], 0.0, 1.0)
    z = jax.lax.dot_general(A2, dq_ref[...], (((1,), (0,)), ((), ())),
                            preferred_element_type=f32) + q0_ref[...]
    delta = z - Xm
    out_ref[...] = x + jax.lax.dot_general(
        delta, wT, (((1,), (1,)), ((), ())), preferred_element_type=f32)


def kernel(X, y, wT, bin_edges, cdf_vals, bary_q):
    f32 = jnp.float32
    N, D = X.shape
    K = wT.shape[1]
    NDOM, _, Bp1 = bin_edges.shape
    B = Bp1 - 1
    G = bary_q.shape[1]
    R = _ROWS
    NB = N // R
    KB = K * B
    KG = K * (G - 1)

    e0 = bin_edges[:, :, :B]
    invw = 1.0 / (bin_edges[:, :, 1:] - e0)
    invw_f = invw.reshape(NDOM, KB)
    esc_f = (e0 * invw).reshape(NDOM, KB)
    df_f = (cdf_vals[:, :, 1:] - cdf_vals[:, :, :B]).reshape(NDOM, KB)

    colk = (jnp.arange(KB) // B)[None, :]
    EREP = (jnp.arange(K)[:, None] == colk).astype(f32)
    SSEG = EREP.T
    colg = jnp.arange(KG) // (G - 1)
    EREP2 = jnp.where(jnp.arange(K)[:, None] == colg[None, :],
                      f32(G - 1), f32(0))
    GIOTA = (jnp.arange(KG) % (G - 1)).astype(f32)[None, :]
    dq = (bary_q[:, 1:] - bary_q[:, :-1]).reshape(KG)
    DQ = jnp.where(colg[:, None] == jnp.arange(K)[None, :],
                   dq[:, None], f32(0))
    Q0 = bary_q[:, 0][None, :]

    y2 = y.astype(jnp.int32).reshape(N, 1)

    out = pl.pallas_call(
        _body,
        grid=(NB,),
        in_specs=[
            pl.BlockSpec((R, D), lambda i: (i, 0)),
            pl.BlockSpec((R, 1), lambda i: (i, 0)),
            pl.BlockSpec((D, K), lambda i: (0, 0)),
            pl.BlockSpec((NDOM, KB), lambda i: (0, 0)),
            pl.BlockSpec((NDOM, KB), lambda i: (0, 0)),
            pl.BlockSpec((NDOM, KB), lambda i: (0, 0)),
            pl.BlockSpec((K, KB), lambda i: (0, 0)),
            pl.BlockSpec((KB, K), lambda i: (0, 0)),
            pl.BlockSpec((K, KG), lambda i: (0, 0)),
            pl.BlockSpec((1, KG), lambda i: (0, 0)),
            pl.BlockSpec((KG, K), lambda i: (0, 0)),
            pl.BlockSpec((1, K), lambda i: (0, 0)),
        ],
        out_specs=pl.BlockSpec((R, D), lambda i: (i, 0)),
        out_shape=jax.ShapeDtypeStruct((N, D), f32),
        compiler_params=pltpu.CompilerParams(
            dimension_semantics=("parallel",),
        ),
    )(X, y2, wT, invw_f, esc_f, df_f, EREP, SSEG, EREP2, GIOTA, DQ, Q0)
    return out


# speedup vs baseline: 72.0598x; 72.0598x over previous
import jax
import jax.numpy as jnp
from jax.experimental import pallas as pl
from jax.experimental.pallas import tpu as pltpu

_ROWS = 512


def _body(x_ref, y_ref, wT_ref, invw_ref, e_ref, df_ref, erep_ref,
          sseg_ref, erep2_ref, giota_ref, dq_ref, q0_ref, out_ref):
    f32 = jnp.float32
    x = x_ref[...]
    wT = wT_ref[...]
    Xm = jax.lax.dot_general(x, wT, (((1,), (0,)), ((), ())),
                             preferred_element_type=f32)
    ndom = invw_ref.shape[0]
    oh = (y_ref[...] == jax.lax.broadcasted_iota(jnp.int32, (1, ndom), 1)
          ).astype(f32)
    xr = jax.lax.dot_general(Xm, erep_ref[...], (((1,), (0,)), ((), ())),
                             preferred_element_type=f32)
    invw_s = jax.lax.dot_general(oh, invw_ref[...], (((1,), (0,)), ((), ())),
                                 preferred_element_type=f32)
    e_s = jax.lax.dot_general(oh, e_ref[...], (((1,), (0,)), ((), ())),
                              preferred_element_type=f32)
    df_s = jax.lax.dot_general(oh, df_ref[...], (((1,), (0,)), ((), ())),
                               preferred_element_type=f32)
    A = jnp.clip((xr - e_s) * invw_s, 0.0, 1.0)
    u = jax.lax.dot_general(df_s * A, sseg_ref[...], (((1,), (0,)), ((), ())),
                            preferred_element_type=f32)
    ur = jax.lax.dot_general(u, erep2_ref[...], (((1,), (0,)), ((), ())),
                             preferred_element_type=f32)
    A2 = jnp.clip(ur - giota_ref[...], 0.0, 1.0)
    z = jax.lax.dot_general(A2, dq_ref[...], (((1,), (0,)), ((), ())),
                            preferred_element_type=f32) + q0_ref[...]
    delta = z - Xm
    out_ref[...] = x + jax.lax.dot_general(
        delta, wT, (((1,), (1,)), ((), ())), preferred_element_type=f32)


def kernel(X, y, wT, bin_edges, cdf_vals, bary_q):
    f32 = jnp.float32
    N, D = X.shape
    K = wT.shape[1]
    NDOM, _, Bp1 = bin_edges.shape
    B = Bp1 - 1
    G = bary_q.shape[1]
    R = _ROWS
    NB = N // R
    KB = K * B
    KG = K * (G - 1)

    e0 = bin_edges[:, :, :B]
    invw = 1.0 / (bin_edges[:, :, 1:] - e0)
    invw_f = invw.reshape(NDOM, KB)
    e_f = e0.reshape(NDOM, KB)
    df_f = (cdf_vals[:, :, 1:] - cdf_vals[:, :, :B]).reshape(NDOM, KB)

    colk = (jnp.arange(KB) // B)[None, :]
    EREP = (jnp.arange(K)[:, None] == colk).astype(f32)
    SSEG = EREP.T
    colg = jnp.arange(KG) // (G - 1)
    EREP2 = jnp.where(jnp.arange(K)[:, None] == colg[None, :],
                      f32(G - 1), f32(0))
    GIOTA = (jnp.arange(KG) % (G - 1)).astype(f32)[None, :]
    dq = (bary_q[:, 1:] - bary_q[:, :-1]).reshape(KG)
    DQ = jnp.where(colg[:, None] == jnp.arange(K)[None, :],
                   dq[:, None], f32(0))
    Q0 = bary_q[:, 0][None, :]

    y2 = y.astype(jnp.int32).reshape(N, 1)

    out = pl.pallas_call(
        _body,
        grid=(NB,),
        in_specs=[
            pl.BlockSpec((R, D), lambda i: (i, 0)),
            pl.BlockSpec((R, 1), lambda i: (i, 0)),
            pl.BlockSpec((D, K), lambda i: (0, 0)),
            pl.BlockSpec((NDOM, KB), lambda i: (0, 0)),
            pl.BlockSpec((NDOM, KB), lambda i: (0, 0)),
            pl.BlockSpec((NDOM, KB), lambda i: (0, 0)),
            pl.BlockSpec((K, KB), lambda i: (0, 0)),
            pl.BlockSpec((KB, K), lambda i: (0, 0)),
            pl.BlockSpec((K, KG), lambda i: (0, 0)),
            pl.BlockSpec((1, KG), lambda i: (0, 0)),
            pl.BlockSpec((KG, K), lambda i: (0, 0)),
            pl.BlockSpec((1, K), lambda i: (0, 0)),
        ],
        out_specs=pl.BlockSpec((R, D), lambda i: (i, 0)),
        out_shape=jax.ShapeDtypeStruct((N, D), f32),
        compiler_params=pltpu.CompilerParams(
            dimension_semantics=("parallel",),
        ),
    )(X, y2, wT, invw_f, e_f, df_f, EREP, SSEG, EREP2, GIOTA, DQ, Q0)
    return out
